# baseline (speedup 1.0000x reference)
import functools

import jax
import jax.numpy as jnp
from jax import lax
from jax.experimental import pallas as pl
from jax.experimental.pallas import tpu as pltpu
from jax.experimental.pallas import tpu_sc as plsc

_SEQ = 2048
_D = 32
_TAB_ROWS = 255
_RPW = 64
_F_ROWS = 4096
_CHUNK = _F_ROWS // 16
_CHUNK_B = _CHUNK + 3
_ROW_W = _SEQ * _D
_FK_W = _F_ROWS * _D
_L = 16
_HALF = _SEQ // 2


def _build_body(table_hbm, f4_hbm, tab_ref, chunk_ref, csem):
    sid = lax.axis_index("s")

    pltpu.sync_copy(table_hbm, tab_ref)

    c254_a = tab_ref[pl.ds(254 * _D, _L)]
    c254_b = tab_ref[pl.ds(254 * _D + _L, _L)]
    c0_a = tab_ref[pl.ds(0, _L)]
    c0_b = tab_ref[pl.ds(_L, _L)]

    m0 = sid * _CHUNK
    m1 = m0 + _CHUNK_B
    e_pre = jnp.clip(1921, m0, m1)
    e_band = jnp.clip(2174, m0, m1)

    def fill_const_pre(m, _):
        o = (m - m0) * _D
        chunk_ref[pl.ds(o, _L)] = c254_a
        chunk_ref[pl.ds(o + _L, _L)] = c254_b
        return _

    def fill_band(m, _):
        o = (m - m0) * _D
        src = (2174 - m) * _D
        chunk_ref[pl.ds(o, _L)] = tab_ref[pl.ds(src, _L)]
        chunk_ref[pl.ds(o + _L, _L)] = tab_ref[pl.ds(src + _L, _L)]
        return _

    def fill_const_post(m, _):
        o = (m - m0) * _D
        chunk_ref[pl.ds(o, _L)] = c0_a
        chunk_ref[pl.ds(o + _L, _L)] = c0_b
        return _

    lax.fori_loop(m0, e_pre, fill_const_pre, 0)
    lax.fori_loop(e_pre, e_band, fill_band, 0)
    lax.fori_loop(e_band, m1, fill_const_post, 0)

    @pl.when(lax.axis_index("c") == 0)
    def _():
        fills = [
            pltpu.async_copy(
                chunk_ref.at[pl.ds(k * _D, _CHUNK * _D)],
                f4_hbm.at[pl.ds(k * _FK_W + m0 * _D, _CHUNK * _D)],
                csem,
            )
            for k in range(4)
        ]
        for cp in fills:
            cp.wait()


def _emit_body(f4_hbm, out_hbm, f_spmem, osem):
    cid = lax.axis_index("c")
    pltpu.sync_copy(f4_hbm, f_spmem)

    i_base = cid * _HALF
    for k in range(4):
        rk = (3 - k) % 4
        src0 = k * _FK_W + 2044 * _D - i_base * _D
        dst0 = (i_base + rk) * _ROW_W

        def fire(q, _, src0=src0, dst0=dst0):
            pltpu.async_copy(
                f_spmem.at[pl.ds(src0 - q * 128, _ROW_W)],
                out_hbm.at[pl.ds(dst0 + q * (4 * _ROW_W), _ROW_W)],
                osem,
            )
            return _

        lax.fori_loop(0, _HALF // 4, fire, 0)

    def drain(q, _):
        pltpu.make_async_copy(
            f4_hbm.at[pl.ds(0, _ROW_W)],
            f_spmem.at[pl.ds(0, _ROW_W)],
            osem,
        ).wait()
        return _

    lax.fori_loop(0, _HALF, drain, 0)


def kernel(seq_len, rel_pos_emb):
    del seq_len
    vmesh = plsc.VectorSubcoreMesh(core_axis_name="c", subcore_axis_name="s")
    build = functools.partial(
        pl.kernel,
        mesh=vmesh,
        out_type=jax.ShapeDtypeStruct((4 * _FK_W,), jnp.float32),
        scratch_types=[
            pltpu.VMEM((_TAB_ROWS * _D,), jnp.float32),
            pltpu.VMEM((_CHUNK_B * _D,), jnp.float32),
            pltpu.SemaphoreType.DMA,
        ],
    )(_build_body)

    smesh = plsc.ScalarSubcoreMesh(axis_name="c", num_cores=2)
    emit = functools.partial(
        pl.kernel,
        mesh=smesh,
        out_type=jax.ShapeDtypeStruct((_SEQ * _SEQ * _D,), jnp.float32),
        scratch_types=[
            pltpu.VMEM_SHARED((4 * _FK_W,), jnp.float32),
            pltpu.SemaphoreType.DMA,
        ],
    )(_emit_body)

    f4 = build(rel_pos_emb.reshape(-1))
    flat = emit(f4)
    return flat.reshape(_SEQ, _SEQ, _D)


# speedup vs baseline: 1.0582x; 1.0582x over previous
import functools

import jax
import jax.numpy as jnp
from jax import lax
from jax.experimental import pallas as pl
from jax.experimental.pallas import tpu as pltpu
from jax.experimental.pallas import tpu_sc as plsc

_SEQ = 2048
_D = 32
_TAB_ROWS = 255
_F_ROWS = 4096
_CHUNK = _F_ROWS // 16
_CHUNK_B = _CHUNK + 3
_ROW_W = _SEQ * _D
_FK_W = _F_ROWS * _D
_L = 16
_VROW = 128
_ROW_VR = _ROW_W // _VROW


def _build_body(table_hbm, f4_hbm, tab_ref, chunk_ref, csem):
    sid = lax.axis_index("s")

    pltpu.sync_copy(table_hbm, tab_ref)

    c254_a = tab_ref[pl.ds(254 * _D, _L)]
    c254_b = tab_ref[pl.ds(254 * _D + _L, _L)]
    c0_a = tab_ref[pl.ds(0, _L)]
    c0_b = tab_ref[pl.ds(_L, _L)]

    m0 = sid * _CHUNK
    m1 = m0 + _CHUNK_B
    e_pre = jnp.clip(1921, m0, m1)
    e_band = jnp.clip(2174, m0, m1)

    def fill_const_pre(m, _):
        o = (m - m0) * _D
        chunk_ref[pl.ds(o, _L)] = c254_a
        chunk_ref[pl.ds(o + _L, _L)] = c254_b
        return _

    def fill_band(m, _):
        o = (m - m0) * _D
        src = (2174 - m) * _D
        chunk_ref[pl.ds(o, _L)] = tab_ref[pl.ds(src, _L)]
        chunk_ref[pl.ds(o + _L, _L)] = tab_ref[pl.ds(src + _L, _L)]
        return _

    def fill_const_post(m, _):
        o = (m - m0) * _D
        chunk_ref[pl.ds(o, _L)] = c0_a
        chunk_ref[pl.ds(o + _L, _L)] = c0_b
        return _

    lax.fori_loop(m0, e_pre, fill_const_pre, 0)
    lax.fori_loop(e_pre, e_band, fill_band, 0)
    lax.fori_loop(e_band, m1, fill_const_post, 0)

    @pl.when(lax.axis_index("c") == 0)
    def _():
        fills = [
            pltpu.async_copy(
                chunk_ref.at[pl.ds(k * _D, _CHUNK * _D)],
                f4_hbm.at[pl.ds(k * _FK_W + m0 * _D, _CHUNK * _D)],
                csem,
            )
            for k in range(4)
        ]
        for cp in fills:
            cp.wait()


def _tc_emit_body(f4_hbm, out_hbm, f_vmem, lsem, osem):
    cp = pltpu.make_async_copy(f4_hbm, f_vmem, lsem)
    cp.start()
    cp.wait()

    def fire(i, _):
        s = 2047 - i
        row = (s % 4) * (_FK_W // _VROW) + s // 4
        pltpu.make_async_copy(
            f_vmem.at[pl.ds(row, _ROW_VR), :],
            out_hbm.at[pl.ds(i * _ROW_VR, _ROW_VR), :],
            osem,
        ).start()
        return _

    lax.fori_loop(0, _SEQ, fire, 0)

    def drain(i, _):
        pltpu.make_async_copy(
            f_vmem.at[pl.ds(0, _ROW_VR), :],
            out_hbm.at[pl.ds(0, _ROW_VR), :],
            osem,
        ).wait()
        return _

    lax.fori_loop(0, _SEQ, drain, 0)


def kernel(seq_len, rel_pos_emb):
    del seq_len
    vmesh = plsc.VectorSubcoreMesh(core_axis_name="c", subcore_axis_name="s")
    build = functools.partial(
        pl.kernel,
        mesh=vmesh,
        out_type=jax.ShapeDtypeStruct((4 * _FK_W,), jnp.float32),
        scratch_types=[
            pltpu.VMEM((_TAB_ROWS * _D,), jnp.float32),
            pltpu.VMEM((_CHUNK_B * _D,), jnp.float32),
            pltpu.SemaphoreType.DMA,
        ],
    )(_build_body)

    emit = pl.pallas_call(
        _tc_emit_body,
        in_specs=[pl.BlockSpec(memory_space=pl.ANY)],
        out_specs=pl.BlockSpec(memory_space=pl.ANY),
        out_shape=jax.ShapeDtypeStruct((_SEQ * _ROW_VR, _VROW), jnp.float32),
        scratch_shapes=[
            pltpu.VMEM((4 * _FK_W // _VROW, _VROW), jnp.float32),
            pltpu.SemaphoreType.DMA,
            pltpu.SemaphoreType.DMA,
        ],
    )

    f4 = build(rel_pos_emb.reshape(-1))
    flat = emit(f4.reshape(4 * _FK_W // _VROW, _VROW))
    return flat.reshape(_SEQ, _SEQ, _D)


# speedup vs baseline: 1.1343x; 1.0719x over previous
import functools

import jax
import jax.numpy as jnp
from jax import lax
from jax.experimental import pallas as pl
from jax.experimental.pallas import tpu as pltpu
from jax.experimental.pallas import tpu_sc as plsc

_SEQ = 2048
_D = 32
_TAB_ROWS = 255
_F_ROWS = 4096
_CHUNK = _F_ROWS // 16
_CHUNK_B = _CHUNK + 7
_ROW_W = _SEQ * _D
_FK_W = _F_ROWS * _D
_L = 16
_VROW = 128
_ROW_VR = _ROW_W // _VROW


def _build_body(table_hbm, f4_hbm, tab_ref, chunk_ref, csem):
    sid = lax.axis_index("s")

    pltpu.sync_copy(table_hbm, tab_ref)

    c254_a = tab_ref[pl.ds(254 * _D, _L)]
    c254_b = tab_ref[pl.ds(254 * _D + _L, _L)]
    c0_a = tab_ref[pl.ds(0, _L)]
    c0_b = tab_ref[pl.ds(_L, _L)]

    m0 = sid * _CHUNK
    m1 = m0 + _CHUNK_B
    e_pre = jnp.clip(1921, m0, m1)
    e_band = jnp.clip(2174, m0, m1)

    def fill_const_pre(m, _):
        o = (m - m0) * _D
        chunk_ref[pl.ds(o, _L)] = c254_a
        chunk_ref[pl.ds(o + _L, _L)] = c254_b
        return _

    def fill_band(m, _):
        o = (m - m0) * _D
        src = (2174 - m) * _D
        chunk_ref[pl.ds(o, _L)] = tab_ref[pl.ds(src, _L)]
        chunk_ref[pl.ds(o + _L, _L)] = tab_ref[pl.ds(src + _L, _L)]
        return _

    def fill_const_post(m, _):
        o = (m - m0) * _D
        chunk_ref[pl.ds(o, _L)] = c0_a
        chunk_ref[pl.ds(o + _L, _L)] = c0_b
        return _

    lax.fori_loop(m0, e_pre, fill_const_pre, 0)
    lax.fori_loop(e_pre, e_band, fill_band, 0)
    lax.fori_loop(e_band, m1, fill_const_post, 0)

    @pl.when(lax.axis_index("c") == 0)
    def _():
        fills = [
            pltpu.async_copy(
                chunk_ref.at[pl.ds(k * _D, _CHUNK * _D)],
                f4_hbm.at[pl.ds(k * _FK_W + m0 * _D, _CHUNK * _D)],
                csem,
            )
            for k in range(8)
        ]
        for cp in fills:
            cp.wait()


def _tc_emit_body(f8_hbm, out_hbm, f_vmem, lsem, osem):
    cp = pltpu.make_async_copy(f8_hbm, f_vmem, lsem)
    cp.start()
    cp.wait()

    def fire(i, _):
        s = 2047 - i
        k = s % 8
        row = k * _F_ROWS + (s - k)
        pltpu.make_async_copy(
            f_vmem.at[pl.ds(row, _SEQ), :],
            out_hbm.at[i],
            osem,
        ).start()
        return _

    lax.fori_loop(0, _SEQ, fire, 0)

    def drain(i, _):
        pltpu.make_async_copy(
            f_vmem.at[pl.ds(0, _SEQ), :],
            out_hbm.at[0],
            osem,
        ).wait()
        return _

    lax.fori_loop(0, _SEQ, drain, 0)


def kernel(seq_len, rel_pos_emb):
    del seq_len
    vmesh = plsc.VectorSubcoreMesh(core_axis_name="c", subcore_axis_name="s")
    build = functools.partial(
        pl.kernel,
        mesh=vmesh,
        out_type=jax.ShapeDtypeStruct((8 * _FK_W,), jnp.float32),
        scratch_types=[
            pltpu.VMEM((_TAB_ROWS * _D,), jnp.float32),
            pltpu.VMEM((_CHUNK_B * _D,), jnp.float32),
            pltpu.SemaphoreType.DMA,
        ],
    )(_build_body)

    emit = pl.pallas_call(
        _tc_emit_body,
        in_specs=[pl.BlockSpec(memory_space=pl.ANY)],
        out_specs=pl.BlockSpec(memory_space=pl.ANY),
        out_shape=jax.ShapeDtypeStruct((_SEQ, _SEQ, _D), jnp.float32),
        scratch_shapes=[
            pltpu.VMEM((8 * _F_ROWS, _D), jnp.float32),
            pltpu.SemaphoreType.DMA,
            pltpu.SemaphoreType.DMA,
        ],
    )

    f8 = build(rel_pos_emb.reshape(-1))
    return emit(f8.reshape(8 * _F_ROWS, _D))
